# baseline (speedup 1.0000x reference)
import functools

import jax
import jax.numpy as jnp
from jax import lax
from jax.experimental import pallas as pl
from jax.experimental.pallas import tpu as pltpu
from jax.experimental.pallas import tpu_sc as plsc

B = 4
N = 1024
K = 20
EPS = 1e-5
RB = 256
NB = N // RB
NP = B * N


def _knn_stage(x, sq, wcT):
    NBAT = x.shape[0]
    C = x.shape[-1]
    O = wcT.shape[-1]

    def kern(xa_ref, xr_ref, sq_ref, wc_ref, idx_ref, c_ref):
        b = pl.program_id(0)
        xa = xa_ref[0]
        xr = xr_ref[0]
        xab = xa.astype(jnp.bfloat16)
        xrb = xr.astype(jnp.bfloat16)
        inner = lax.dot_general(xrb, xab, (((1,), (1,)), ((), ())),
                                preferred_element_type=jnp.float32)
        sqr = jnp.sum(xr * xr, axis=1, keepdims=True)
        score = (sqr - 2.0 * inner) + sq_ref[0]
        iota = lax.broadcasted_iota(jnp.int32, (RB, N), 1)
        kiota = lax.broadcasted_iota(jnp.int32, (RB, K), 1)
        ki = lax.bitcast_convert_type(score + 1.0, jnp.int32)
        key = lax.bitcast_convert_type((ki & (-1024)) | iota, jnp.float32)
        acc = jnp.zeros((RB, K), jnp.int32)
        for k in range(K):
            mk = jnp.min(key, axis=1, keepdims=True)
            minidx = lax.bitcast_convert_type(mk, jnp.int32) & 1023
            acc = jnp.where(kiota == k, minidx, acc)
            key = jnp.where(key == mk, jnp.inf, key)
        idx_ref[0] = acc + b * N
        c_ref[0] = jnp.dot(xrb, wc_ref[...].astype(jnp.bfloat16),
                           preferred_element_type=jnp.float32)

    return pl.pallas_call(
        kern,
        grid=(NBAT, NB),
        in_specs=[
            pl.BlockSpec((1, N, C), lambda b, r: (b, 0, 0)),
            pl.BlockSpec((1, RB, C), lambda b, r: (b, r, 0)),
            pl.BlockSpec((1, 1, N), lambda b, r: (b, 0, 0)),
            pl.BlockSpec((C, O), lambda b, r: (0, 0)),
        ],
        out_specs=[
            pl.BlockSpec((1, RB, K), lambda b, r: (b, r, 0)),
            pl.BlockSpec((1, RB, O), lambda b, r: (b, r, 0)),
        ],
        out_shape=[
            jax.ShapeDtypeStruct((NBAT, N, K), jnp.int32),
            jax.ShapeDtypeStruct((NBAT, N, O), jnp.float32),
        ],
    )(x, x, sq, wcT)


def _gather_rows(x_pad, idx2):
    CP = x_pad.shape[1]
    CHW = idx2.shape[1]
    TR = idx2.shape[0] * CHW
    info = plsc.get_sparse_core_info()
    nw = info.num_cores * info.num_subcores
    nch = TR // (nw * CHW)
    npair = nch // 2
    mesh = plsc.VectorSubcoreMesh(core_axis_name="c", subcore_axis_name="s")

    @functools.partial(
        pl.kernel, mesh=mesh,
        out_type=jax.ShapeDtypeStruct((TR, CP), jnp.float32),
        scratch_types=[
            pltpu.VMEM((nch, CHW), jnp.int32),
            pltpu.VMEM((CHW, CP), jnp.float32),
            pltpu.VMEM((CHW, CP), jnp.float32),
            pltpu.SemaphoreType.DMA,
            pltpu.SemaphoreType.DMA,
        ],
    )
    def gk(x_hbm, idx_hbm, out_hbm, idx_v, r0, r1, sem0, sem1):
        wid = lax.axis_index("s") * info.num_cores + lax.axis_index("c")
        cbase = wid * nch
        pltpu.sync_copy(idx_hbm.at[pl.ds(cbase, nch)], idx_v)
        pltpu.make_async_copy(x_hbm.at[idx_v.at[0]], r0, sem0).start()

        def body(pair, carry):
            c0 = 2 * pair
            c1 = c0 + 1
            pltpu.make_async_copy(x_hbm.at[idx_v.at[c1]], r1, sem1).start()
            pltpu.make_async_copy(x_hbm.at[idx_v.at[c0]], r0, sem0).wait()
            pltpu.sync_copy(r0, out_hbm.at[pl.ds((cbase + c0) * CHW, CHW)])

            @pl.when(pair + 1 < npair)
            def _():
                pltpu.make_async_copy(x_hbm.at[idx_v.at[c0 + 2]], r0,
                                      sem0).start()

            pltpu.make_async_copy(x_hbm.at[idx_v.at[c1]], r1, sem1).wait()
            pltpu.sync_copy(r1, out_hbm.at[pl.ds((cbase + c1) * CHW, CHW)])
            return carry

        lax.fori_loop(0, npair, body, 0)

    return gk(x_pad, idx2)


def _edge_conv(xg, x, c, wdT, s_row, t_row):
    NBAT = x.shape[0]
    C = x.shape[-1]
    O = wdT.shape[-1]
    CP = xg.shape[-1] // K

    def kern(xg_ref, xr_ref, c_ref, wd_ref, s_ref, t_ref, out_ref):
        xr = xr_ref[0]
        wd = wd_ref[...].astype(jnp.bfloat16)
        xg2 = xg_ref[0]
        m = None
        for k in range(K):
            dk = xg2[:, k * CP:k * CP + C] - xr
            yk = jnp.dot(dk.astype(jnp.bfloat16), wd,
                         preferred_element_type=jnp.float32)
            m = yk if m is None else jnp.maximum(m, yk)
        y = s_ref[...] * (m + c_ref[0]) + t_ref[...]
        out_ref[0] = jnp.where(y > 0, y, 0.2 * y)

    return pl.pallas_call(
        kern,
        grid=(NBAT, NB),
        in_specs=[
            pl.BlockSpec((1, RB, K * CP), lambda b, r: (b, r, 0)),
            pl.BlockSpec((1, RB, C), lambda b, r: (b, r, 0)),
            pl.BlockSpec((1, RB, O), lambda b, r: (b, r, 0)),
            pl.BlockSpec((C, O), lambda b, r: (0, 0)),
            pl.BlockSpec((1, O), lambda b, r: (0, 0)),
            pl.BlockSpec((1, O), lambda b, r: (0, 0)),
        ],
        out_specs=pl.BlockSpec((1, RB, O), lambda b, r: (b, r, 0)),
        out_shape=jax.ShapeDtypeStruct((NBAT, N, O), jnp.float32),
    )(xg, x, c, wdT, s_row, t_row)


def _final_stage(x1, x2, x3, x4, w1T, w2T, w3T, w4T, s_row, t_row):
    def kern(x1_ref, x2_ref, x3_ref, x4_ref, w1_ref, w2_ref, w3_ref, w4_ref,
             s_ref, t_ref, out_ref):
        bf = jnp.bfloat16
        y = jnp.dot(x1_ref[0].astype(bf), w1_ref[...].astype(bf),
                    preferred_element_type=jnp.float32)
        y += jnp.dot(x2_ref[0].astype(bf), w2_ref[...].astype(bf),
                     preferred_element_type=jnp.float32)
        y += jnp.dot(x3_ref[0].astype(bf), w3_ref[...].astype(bf),
                     preferred_element_type=jnp.float32)
        y += jnp.dot(x4_ref[0].astype(bf), w4_ref[...].astype(bf),
                     preferred_element_type=jnp.float32)
        y = y * s_ref[...] + t_ref[...]
        act = jnp.where(y > 0, y, 0.2 * y)
        out_ref[0, 0, :1024] = jnp.max(act, axis=0)
        out_ref[0, 0, 1024:] = jnp.sum(act, axis=0) * (1.0 / N)

    o1, o2, o3, o4 = (a.shape[-1] for a in (x1, x2, x3, x4))
    return pl.pallas_call(
        kern,
        grid=(B,),
        in_specs=[
            pl.BlockSpec((1, N, o1), lambda b: (b, 0, 0)),
            pl.BlockSpec((1, N, o2), lambda b: (b, 0, 0)),
            pl.BlockSpec((1, N, o3), lambda b: (b, 0, 0)),
            pl.BlockSpec((1, N, o4), lambda b: (b, 0, 0)),
            pl.BlockSpec((o1, 1024), lambda b: (0, 0)),
            pl.BlockSpec((o2, 1024), lambda b: (0, 0)),
            pl.BlockSpec((o3, 1024), lambda b: (0, 0)),
            pl.BlockSpec((o4, 1024), lambda b: (0, 0)),
            pl.BlockSpec((1, 1024), lambda b: (0, 0)),
            pl.BlockSpec((1, 1024), lambda b: (0, 0)),
        ],
        out_specs=pl.BlockSpec((1, 1, 2048), lambda b: (b, 0, 0)),
        out_shape=jax.ShapeDtypeStruct((B, 1, 2048), jnp.float32),
    )(x1, x2, x3, x4, w1T, w2T, w3T, w4T, s_row, t_row)


def _bn_fold(g, b, m, v):
    s = g / jnp.sqrt(v + EPS)
    return s, b - m * s


def kernel(x, W1, g1, b1, m1, v1, W2, g2, b2, m2, v2, W3, g3, b3, m3, v3,
           W4, g4, b4, m4, v4, W5, g5, b5, m5, v5):
    chains = [x[b:b + 1] for b in range(B)]
    feats = [[] for _ in range(B)]
    for (W, g, bb, m, v) in ((W1, g1, b1, m1, v1), (W2, g2, b2, m2, v2),
                             (W3, g3, b3, m3, v3), (W4, g4, b4, m4, v4)):
        s, t = _bn_fold(g, bb, m, v)
        for b in range(B):
            xc = chains[b]
            C = xc.shape[-1]
            O = W.shape[0]
            wdT = W[:, :C].T
            wcT = W[:, C:].T
            sq = jnp.sum(xc * xc, axis=-1).reshape(1, 1, N)
            idx, c = _knn_stage(xc, sq, wcT)
            CP = max(C, 128)
            x_pad = jnp.pad(xc.reshape(N, C), ((0, 0), (0, CP - C)))
            xg = _gather_rows(x_pad, idx.reshape(N * K // 80, 80))
            xc = _edge_conv(xg.reshape(1, N, K * CP), xc, c, wdT,
                            s.reshape(1, O), t.reshape(1, O))
            chains[b] = xc
            feats[b].append(xc)

    cat = [jnp.concatenate([feats[b][i] for b in range(B)], axis=0)
           for i in range(4)]

    s5, t5 = _bn_fold(g5, b5, m5, v5)
    offs = (0, 64, 128, 256, 512)
    wT = [W5[:, offs[i]:offs[i + 1]].T for i in range(4)]
    return _final_stage(cat[0], cat[1], cat[2], cat[3],
                        wT[0], wT[1], wT[2], wT[3], s5.reshape(1, 1024),
                        t5.reshape(1, 1024)).reshape(B, 2048)


# speedup vs baseline: 1.1348x; 1.1348x over previous
import functools

import jax
import jax.numpy as jnp
from jax import lax
from jax.experimental import pallas as pl
from jax.experimental.pallas import tpu as pltpu
from jax.experimental.pallas import tpu_sc as plsc

B = 4
N = 1024
K = 20
EPS = 1e-5
RB = 256
NB = N // RB
NP = B * N


def _knn_stage(x, sq, wcT):
    NBAT = x.shape[0]
    C = x.shape[-1]
    O = wcT.shape[-1]

    def kern(xa_ref, xr_ref, sq_ref, wc_ref, idx_ref, c_ref):
        b = pl.program_id(0)
        xa = xa_ref[0]
        xr = xr_ref[0]
        xab = xa.astype(jnp.bfloat16)
        xrb = xr.astype(jnp.bfloat16)
        inner = lax.dot_general(xrb, xab, (((1,), (1,)), ((), ())),
                                preferred_element_type=jnp.float32)
        sqr = jnp.sum(xr * xr, axis=1, keepdims=True)
        score = (sqr - 2.0 * inner) + sq_ref[0]
        iota = lax.broadcasted_iota(jnp.int32, (RB, N), 1)
        kiota = lax.broadcasted_iota(jnp.int32, (RB, K), 1)
        ki = lax.bitcast_convert_type(score + 1.0, jnp.int32)
        key = lax.bitcast_convert_type((ki & (-1024)) | iota, jnp.float32)
        acc = jnp.zeros((RB, K), jnp.int32)
        for k in range(K):
            mk = jnp.min(key, axis=1, keepdims=True)
            minidx = lax.bitcast_convert_type(mk, jnp.int32) & 1023
            acc = jnp.where(kiota == k, minidx, acc)
            key = jnp.where(key == mk, jnp.inf, key)
        idx_ref[0] = acc + b * N
        c_ref[0] = jnp.dot(xrb, wc_ref[...].astype(jnp.bfloat16),
                           preferred_element_type=jnp.float32)

    return pl.pallas_call(
        kern,
        grid=(NBAT, NB),
        in_specs=[
            pl.BlockSpec((1, N, C), lambda b, r: (b, 0, 0)),
            pl.BlockSpec((1, RB, C), lambda b, r: (b, r, 0)),
            pl.BlockSpec((1, 1, N), lambda b, r: (b, 0, 0)),
            pl.BlockSpec((C, O), lambda b, r: (0, 0)),
        ],
        out_specs=[
            pl.BlockSpec((1, RB, K), lambda b, r: (b, r, 0)),
            pl.BlockSpec((1, RB, O), lambda b, r: (b, r, 0)),
        ],
        out_shape=[
            jax.ShapeDtypeStruct((NBAT, N, K), jnp.int32),
            jax.ShapeDtypeStruct((NBAT, N, O), jnp.float32),
        ],
    )(x, x, sq, wcT)


def _gather_rows(x_pad, idx2):
    CP = x_pad.shape[1]
    CHW = idx2.shape[1]
    TR = idx2.shape[0] * CHW
    info = plsc.get_sparse_core_info()
    nw = info.num_cores * info.num_subcores
    nch = TR // (nw * CHW)
    npair = nch // 2
    mesh = plsc.VectorSubcoreMesh(core_axis_name="c", subcore_axis_name="s")

    @functools.partial(
        pl.kernel, mesh=mesh,
        out_type=jax.ShapeDtypeStruct((TR, CP), jnp.float32),
        scratch_types=[
            pltpu.VMEM((nch, CHW), jnp.int32),
            pltpu.VMEM((CHW, CP), jnp.float32),
            pltpu.VMEM((CHW, CP), jnp.float32),
            pltpu.SemaphoreType.DMA,
            pltpu.SemaphoreType.DMA,
        ],
    )
    def gk(x_hbm, idx_hbm, out_hbm, idx_v, r0, r1, sem0, sem1):
        wid = lax.axis_index("s") * info.num_cores + lax.axis_index("c")
        cbase = wid * nch
        pltpu.sync_copy(idx_hbm.at[pl.ds(cbase, nch)], idx_v)
        pltpu.make_async_copy(x_hbm.at[idx_v.at[0]], r0, sem0).start()

        def body(pair, carry):
            c0 = 2 * pair
            c1 = c0 + 1
            pltpu.make_async_copy(x_hbm.at[idx_v.at[c1]], r1, sem1).start()
            pltpu.make_async_copy(x_hbm.at[idx_v.at[c0]], r0, sem0).wait()
            pltpu.sync_copy(r0, out_hbm.at[pl.ds((cbase + c0) * CHW, CHW)])

            @pl.when(pair + 1 < npair)
            def _():
                pltpu.make_async_copy(x_hbm.at[idx_v.at[c0 + 2]], r0,
                                      sem0).start()

            pltpu.make_async_copy(x_hbm.at[idx_v.at[c1]], r1, sem1).wait()
            pltpu.sync_copy(r1, out_hbm.at[pl.ds((cbase + c1) * CHW, CHW)])
            return carry

        lax.fori_loop(0, npair, body, 0)

    return gk(x_pad, idx2)


def _edge_conv(xg, x, c, wdT, s_row, t_row):
    NBAT = x.shape[0]
    C = x.shape[-1]
    O = wdT.shape[-1]
    CP = xg.shape[-1] // K

    def kern(xg_ref, xr_ref, c_ref, wd_ref, s_ref, t_ref, out_ref):
        xr = xr_ref[0]
        wd = wd_ref[...].astype(jnp.bfloat16)
        xg2 = xg_ref[0]
        m = None
        for k in range(K):
            dk = xg2[:, k * CP:k * CP + C] - xr
            yk = jnp.dot(dk.astype(jnp.bfloat16), wd,
                         preferred_element_type=jnp.float32)
            m = yk if m is None else jnp.maximum(m, yk)
        y = s_ref[...] * (m + c_ref[0]) + t_ref[...]
        out_ref[0] = jnp.where(y > 0, y, 0.2 * y)

    return pl.pallas_call(
        kern,
        grid=(NBAT, NB),
        in_specs=[
            pl.BlockSpec((1, RB, K * CP), lambda b, r: (b, r, 0)),
            pl.BlockSpec((1, RB, C), lambda b, r: (b, r, 0)),
            pl.BlockSpec((1, RB, O), lambda b, r: (b, r, 0)),
            pl.BlockSpec((C, O), lambda b, r: (0, 0)),
            pl.BlockSpec((1, O), lambda b, r: (0, 0)),
            pl.BlockSpec((1, O), lambda b, r: (0, 0)),
        ],
        out_specs=pl.BlockSpec((1, RB, O), lambda b, r: (b, r, 0)),
        out_shape=jax.ShapeDtypeStruct((NBAT, N, O), jnp.float32),
    )(xg, x, c, wdT, s_row, t_row)


def _final_stage(x1, x2, x3, x4, w1T, w2T, w3T, w4T, s_row, t_row):
    def kern(x1_ref, x2_ref, x3_ref, x4_ref, w1_ref, w2_ref, w3_ref, w4_ref,
             s_ref, t_ref, out_ref):
        bf = jnp.bfloat16
        y = jnp.dot(x1_ref[0].astype(bf), w1_ref[...].astype(bf),
                    preferred_element_type=jnp.float32)
        y += jnp.dot(x2_ref[0].astype(bf), w2_ref[...].astype(bf),
                     preferred_element_type=jnp.float32)
        y += jnp.dot(x3_ref[0].astype(bf), w3_ref[...].astype(bf),
                     preferred_element_type=jnp.float32)
        y += jnp.dot(x4_ref[0].astype(bf), w4_ref[...].astype(bf),
                     preferred_element_type=jnp.float32)
        y = y * s_ref[...] + t_ref[...]
        act = jnp.where(y > 0, y, 0.2 * y)
        out_ref[0, 0, :1024] = jnp.max(act, axis=0)
        out_ref[0, 0, 1024:] = jnp.sum(act, axis=0) * (1.0 / N)

    o1, o2, o3, o4 = (a.shape[-1] for a in (x1, x2, x3, x4))
    return pl.pallas_call(
        kern,
        grid=(B,),
        in_specs=[
            pl.BlockSpec((1, N, o1), lambda b: (b, 0, 0)),
            pl.BlockSpec((1, N, o2), lambda b: (b, 0, 0)),
            pl.BlockSpec((1, N, o3), lambda b: (b, 0, 0)),
            pl.BlockSpec((1, N, o4), lambda b: (b, 0, 0)),
            pl.BlockSpec((o1, 1024), lambda b: (0, 0)),
            pl.BlockSpec((o2, 1024), lambda b: (0, 0)),
            pl.BlockSpec((o3, 1024), lambda b: (0, 0)),
            pl.BlockSpec((o4, 1024), lambda b: (0, 0)),
            pl.BlockSpec((1, 1024), lambda b: (0, 0)),
            pl.BlockSpec((1, 1024), lambda b: (0, 0)),
        ],
        out_specs=pl.BlockSpec((1, 1, 2048), lambda b: (b, 0, 0)),
        out_shape=jax.ShapeDtypeStruct((B, 1, 2048), jnp.float32),
    )(x1, x2, x3, x4, w1T, w2T, w3T, w4T, s_row, t_row)


def _bn_fold(g, b, m, v):
    s = g / jnp.sqrt(v + EPS)
    return s, b - m * s


def kernel(x, W1, g1, b1, m1, v1, W2, g2, b2, m2, v2, W3, g3, b3, m3, v3,
           W4, g4, b4, m4, v4, W5, g5, b5, m5, v5):
    NCH = 2
    BPC = B // NCH
    chains = [x[i * BPC:(i + 1) * BPC] for i in range(NCH)]
    feats = [[] for _ in range(NCH)]
    for (W, g, bb, m, v) in ((W1, g1, b1, m1, v1), (W2, g2, b2, m2, v2),
                             (W3, g3, b3, m3, v3), (W4, g4, b4, m4, v4)):
        s, t = _bn_fold(g, bb, m, v)
        for ci in range(NCH):
            xc = chains[ci]
            C = xc.shape[-1]
            O = W.shape[0]
            wdT = W[:, :C].T
            wcT = W[:, C:].T
            sq = jnp.sum(xc * xc, axis=-1).reshape(BPC, 1, N)
            idx, c = _knn_stage(xc, sq, wcT)
            CP = max(C, 128)
            x_pad = jnp.pad(xc.reshape(BPC * N, C), ((0, 0), (0, CP - C)))
            xg = _gather_rows(x_pad, idx.reshape(BPC * N * K // 80, 80))
            xc = _edge_conv(xg.reshape(BPC, N, K * CP), xc, c, wdT,
                            s.reshape(1, O), t.reshape(1, O))
            chains[ci] = xc
            feats[ci].append(xc)

    cat = [jnp.concatenate([feats[ci][i] for ci in range(NCH)], axis=0)
           for i in range(4)]

    s5, t5 = _bn_fold(g5, b5, m5, v5)
    offs = (0, 64, 128, 256, 512)
    wT = [W5[:, offs[i]:offs[i + 1]].T for i in range(4)]
    return _final_stage(cat[0], cat[1], cat[2], cat[3],
                        wT[0], wT[1], wT[2], wT[3], s5.reshape(1, 1024),
                        t5.reshape(1, 1024)).reshape(B, 2048)
